# baseline (speedup 1.0000x reference)
import jax
import jax.numpy as jnp
from jax import lax
from jax.experimental import pallas as pl
from jax.experimental.pallas import tpu as pltpu
from jax.experimental.pallas import tpu_sc as plsc

N = 10000
E = 160000
FEAT = 256
HALF = 128
NSEG = 64
NC = 2
NS = 16
LANES = 16

_MESH = dict(core_axis_name="c", subcore_axis_name="s", num_cores=NC,
             num_subcores=NS)

_Z_CHUNK = 200
_Z_NCHUNK = N // _Z_CHUNK


def _foreach_row_chunk(sid, fn):
  def _j(j, _):
    c = sid + NS * j
    @pl.when(c < _Z_NCHUNK)
    def _():
      fn(c * _Z_CHUNK)
    return _
  lax.fori_loop(0, (_Z_NCHUNK + NS - 1) // NS, _j, None)


_G_CHUNK = 80
_G_NCHUNK = N // _G_CHUNK


_A_CHUNK = 128
_A_NCHUNK = E // _A_CHUNK
_A_NFULL = _A_NCHUNK // NS
_A_NTAIL = _A_NCHUNK - _A_NFULL * NS


def _make_agg(fused_gather=False):
  mesh = plsc.VectorSubcoreMesh(**_MESH)

  def body(*refs):
    if fused_gather:
      (emb_lo, emb_hi, x, edge_index, h_lo, h_hi, s_lo, s_hi,
       gidx, eb0, eb1, eb2, rows0, rows1, rows2, acc,
       isem0, isem1, isem2, gsem0, gsem1, gsem2,
       ssem0, ssem1, ssem2) = refs
    else:
      (t_lo, t_hi, edge_index, s_lo, s_hi,
       eb0, eb1, eb2, rows0, rows1, rows2, acc,
       isem0, isem1, isem2, gsem0, gsem1, gsem2,
       ssem0, ssem1, ssem2) = refs
    ebufs = (eb0, eb1, eb2)
    rows = (rows0, rows1, rows2)
    isems = (isem0, isem1, isem2)
    gsems = (gsem0, gsem1, gsem2)
    ssems = (ssem0, ssem1, ssem2)

    cid = lax.axis_index("c")
    sid = lax.axis_index("s")

    if fused_gather:
      def _do_gather(table, out):
        def _g(j, _):
          chunk = sid * 8 + j
          @pl.when(chunk < _G_NCHUNK)
          def _():
            start = chunk * _G_CHUNK
            pltpu.sync_copy(x.at[pl.ds(start, _G_CHUNK)], gidx)
            pltpu.async_copy(table.at[gidx],
                             rows0.at[pl.ds(0, _G_CHUNK)], gsem0).wait()
            pltpu.sync_copy(rows0.at[pl.ds(0, _G_CHUNK)],
                            out.at[pl.ds(start, _G_CHUNK)])
          return _
        lax.fori_loop(0, 8, _g, None)

      @pl.when(cid == 0)
      def _():
        _do_gather(emb_lo, h_lo)

      @pl.when(cid == 1)
      def _():
        _do_gather(emb_hi, h_hi)
      t_lo, t_hi = h_lo, h_hi

    def _z(i, _):
      def _zz(k, __):
        rows0[i, pl.ds(k * LANES, LANES)] = jnp.zeros((LANES,), jnp.float32)
        return __
      lax.fori_loop(0, HALF // LANES, _zz, None)
      return _
    lax.fori_loop(0, 80, _z, None)

    def _j(j, _):
      c = sid + NS * j
      @pl.when(c < N // 80)
      def _():
        pltpu.sync_copy(rows0.at[pl.ds(0, 80)], acc.at[pl.ds(c * 80, 80)])
      return _
    lax.fori_loop(0, (N // 80 + NS - 1) // NS, _j, None)

    plsc.subcore_barrier()

    def _run(table):
      def _issue_idx(j, k):
        st = (sid + NS * j) * _A_CHUNK
        pltpu.async_copy(edge_index.at[:, pl.ds(st, _A_CHUNK)],
                         ebufs[k], isems[k])

      def _issue_gather(j, k):
        pltpu.async_copy(table.at[ebufs[k].at[0]], rows[k], gsems[k])

      def _wait_idx(k):
        pltpu.make_async_copy(
            edge_index.at[:, pl.ds(0, _A_CHUNK)], ebufs[k], isems[k]).wait()

      def _wait_gather(table, k):
        pltpu.make_async_copy(table.at[ebufs[k].at[0]], rows[k],
                              gsems[k]).wait()

      def _issue_scatter(k):
        pltpu.async_copy(rows[k], acc.at[ebufs[k].at[1]], ssems[k], add=True)

      def _wait_scatter(k):
        pltpu.make_async_copy(rows[k], acc.at[ebufs[k].at[1]],
                              ssems[k]).wait()

      _issue_idx(0, 0)
      _wait_idx(0)
      _issue_gather(0, 0)
      _issue_idx(1, 1)

      def _step(jj, _):
        for k in range(3):
          j = 3 * jj + k
          _wait_gather(table, k)
          _issue_scatter(k)
          if k == 0:
            @pl.when(jj > 0)
            def _():
              _wait_scatter(2)
          else:
            _wait_scatter(k - 1)
          @pl.when(j + 2 < _A_NFULL)
          def _():
            _issue_idx(j + 2, (k + 2) % 3)
          @pl.when(j + 1 < _A_NFULL)
          def _():
            _wait_idx((k + 1) % 3)
            _issue_gather(j + 1, (k + 1) % 3)
        return _
      lax.fori_loop(0, _A_NFULL // 3, _step, None)
      _wait_scatter((_A_NFULL - 1) % 3)

      @pl.when(sid < _A_NTAIL)
      def _():
        st = (_A_NFULL * NS + sid) * _A_CHUNK
        pltpu.async_copy(edge_index.at[:, pl.ds(st, _A_CHUNK)],
                         ebufs[0], isems[0])
        _wait_idx(0)
        _issue_gather(0, 0)
        _wait_gather(table, 0)
        pltpu.sync_copy(rows[0], acc.at[ebufs[0].at[1]], add=True)

    @pl.when(cid == 0)
    def _():
      _run(t_lo)

    @pl.when(cid == 1)
    def _():
      _run(t_hi)

    plsc.subcore_barrier()

    @pl.when(cid == 0)
    def _():
      _foreach_row_chunk(sid, lambda st: pltpu.sync_copy(
          acc.at[pl.ds(st, _Z_CHUNK)], s_lo.at[pl.ds(st, _Z_CHUNK)]))
    @pl.when(cid == 1)
    def _():
      _foreach_row_chunk(sid, lambda st: pltpu.sync_copy(
          acc.at[pl.ds(st, _Z_CHUNK)], s_hi.at[pl.ds(st, _Z_CHUNK)]))

  n_out = 4 if fused_gather else 2
  scratch = ([pltpu.VMEM((_G_CHUNK,), jnp.int32)] if fused_gather else []) + [
      pltpu.VMEM((2, _A_CHUNK), jnp.int32),
      pltpu.VMEM((2, _A_CHUNK), jnp.int32),
      pltpu.VMEM((2, _A_CHUNK), jnp.int32),
      pltpu.VMEM((_A_CHUNK, HALF), jnp.float32),
      pltpu.VMEM((_A_CHUNK, HALF), jnp.float32),
      pltpu.VMEM((_A_CHUNK, HALF), jnp.float32),
      pltpu.VMEM_SHARED((N, HALF), jnp.float32),
  ] + [pltpu.SemaphoreType.DMA] * 9
  return pl.kernel(
      body,
      out_type=[jax.ShapeDtypeStruct((N, HALF), jnp.float32)] * n_out,
      mesh=mesh,
      scratch_types=scratch,
  )


_C_PER_W = E // (NC * NS)


def _make_count():
  mesh = plsc.VectorSubcoreMesh(**_MESH)

  def body(dst, cnt_out, dbuf, hist, sem):
    cid = lax.axis_index("c")
    sid = lax.axis_index("s")
    wid = cid * NS + sid

    def _zh(i, _):
      hist[pl.ds(i * LANES, LANES)] = jnp.zeros((LANES,), jnp.float32)
      return _
    lax.fori_loop(0, N // LANES, _zh, None)

    dbuf[pl.ds(_C_PER_W - 8, LANES)] = jnp.zeros((LANES,), jnp.int32)
    pltpu.sync_copy(dst.at[pl.ds(wid * _C_PER_W, _C_PER_W)],
                    dbuf.at[pl.ds(0, _C_PER_W)])

    ones16 = jnp.ones((LANES,), jnp.float32)

    def _h(i, _):
      dvec = dbuf[pl.ds(i * LANES, LANES)]
      plsc.addupdate_scatter(hist, [dvec], ones16)
      return _
    lax.fori_loop(0, _C_PER_W // LANES, _h, None)
    tvec = dbuf[pl.ds(_C_PER_W - 8, LANES)]
    mask = lax.iota(jnp.int32, LANES) < 8
    plsc.addupdate_scatter(hist, [tvec], ones16, mask=mask)

    pltpu.sync_copy(hist, cnt_out.at[pl.ds(wid * N, N)])

  return pl.kernel(
      body,
      out_type=jax.ShapeDtypeStruct((NC * NS * N,), jnp.float32),
      mesh=mesh,
      scratch_types=[
          pltpu.VMEM((_C_PER_W + 8,), jnp.int32),
          pltpu.VMEM((N,), jnp.float32),
          pltpu.SemaphoreType.DMA,
      ],
      compiler_params=pltpu.CompilerParams(needs_layout_passes=False),
  )


_S_TILE = 400
_S_GRID = N // _S_TILE


def _segmax_body(b, lo, hi, out):
  i = pl.program_id(0)

  @pl.when(i == 0)
  def _():
    out[...] = jnp.full((NSEG, FEAT), -jnp.inf, jnp.float32)

  bb = b[...]
  hc = jnp.concatenate([lo[...], hi[...]], axis=1)
  smin = jnp.min(bb)
  smax = jnp.max(bb)
  for s in range(NSEG):
    @pl.when(jnp.logical_and(smin <= s, s <= smax))
    def _():
      m = jnp.where(bb == s, hc, -jnp.inf)
      v = jnp.max(m, axis=0, keepdims=True)
      out[pl.ds(s, 1), :] = jnp.maximum(out[pl.ds(s, 1), :], v)


def _segmax_tc(batch2d, lo, hi):
  half_spec = pl.BlockSpec((_S_TILE, HALF), lambda i: (i, 0))
  return pl.pallas_call(
      _segmax_body,
      grid=(_S_GRID,),
      in_specs=[
          pl.BlockSpec((_S_TILE, 1), lambda i: (i, 0)),
          half_spec, half_spec,
      ],
      out_specs=pl.BlockSpec((NSEG, FEAT), lambda i: (0, 0)),
      out_shape=jax.ShapeDtypeStruct((NSEG, FEAT), jnp.float32),
  )(batch2d, lo, hi)


_R_TILE = 400
_R_GRID = N // _R_TILE


def _bdot(a, b):
  return lax.dot_general(a.astype(jnp.bfloat16), b.astype(jnp.bfloat16),
                         (((1,), (1,)), ((), ())),
                         preferred_element_type=jnp.float32)


def _sage_dots(m_lo, m_hi, t_lo, t_hi, wlv, wrv):
  acc = _bdot(m_lo, wlv[:, :HALF])
  acc += _bdot(m_hi, wlv[:, HALF:])
  acc += _bdot(t_lo, wrv[:, :HALF])
  acc += _bdot(t_hi, wrv[:, HALF:])
  return acc


def _sage_linear_body(s_lo, s_hi, t_lo, t_hi, cnt, wl, wr, bl,
                      out_lo, out_hi):
  c = lax.dot_general(cnt[...], jnp.ones((NC * NS, 1), jnp.float32),
                      (((1,), (0,)), ((), ())),
                      preferred_element_type=jnp.float32)
  inv = 1.0 / jnp.maximum(c, 1.0)
  acc = _sage_dots(s_lo[...] * inv, s_hi[...] * inv, t_lo[...], t_hi[...],
                   wl[...], wr[...]) + bl[...]
  out_lo[...] = acc[:, :HALF]
  out_hi[...] = acc[:, HALF:]


def _sage_linear(s_lo, s_hi, t_lo, t_hi, cnt, wl, wr, bl):
  half_spec = pl.BlockSpec((_R_TILE, HALF), lambda i: (i, 0))
  return pl.pallas_call(
      _sage_linear_body,
      grid=(_R_GRID,),
      in_specs=[
          half_spec, half_spec, half_spec, half_spec,
          pl.BlockSpec((_R_TILE, NC * NS), lambda i: (i, 0)),
          pl.BlockSpec((FEAT, FEAT), lambda i: (0, 0)),
          pl.BlockSpec((FEAT, FEAT), lambda i: (0, 0)),
          pl.BlockSpec((1, FEAT), lambda i: (0, 0)),
      ],
      out_specs=[half_spec, half_spec],
      out_shape=[
          jax.ShapeDtypeStruct((N, HALF), jnp.float32),
          jax.ShapeDtypeStruct((N, HALF), jnp.float32),
      ],
  )(s_lo, s_hi, t_lo, t_hi, cnt, wl, wr, bl.reshape(1, FEAT))


def _tail_body(b, s_lo, s_hi, t_lo, t_hi, cnt, wl, wr, bl,
               g1, w1, b1, w2, b2, out, g2acc):
  i = pl.program_id(0)

  @pl.when(i == 0)
  def _():
    g2acc[...] = jnp.full((NSEG, FEAT), -jnp.inf, jnp.float32)

  c = lax.dot_general(cnt[...], jnp.ones((NC * NS, 1), jnp.float32),
                      (((1,), (0,)), ((), ())),
                      preferred_element_type=jnp.float32)
  inv = 1.0 / jnp.maximum(c, 1.0)
  dn = (((1,), (1,)), ((), ()))
  acc = _sage_dots(s_lo[...] * inv, s_hi[...] * inv, t_lo[...], t_hi[...],
                   wl[...], wr[...])
  acc += bl[...]

  bb = b[...]
  smin = jnp.min(bb)
  smax = jnp.max(bb)
  for s in range(NSEG):
    @pl.when(jnp.logical_and(smin <= s, s <= smax))
    def _():
      m = jnp.where(bb == s, acc, -jnp.inf)
      v = jnp.max(m, axis=0, keepdims=True)
      g2acc[pl.ds(s, 1), :] = jnp.maximum(g2acc[pl.ds(s, 1), :], v)

  @pl.when(i == _R_GRID - 1)
  def _():
    z = lax.dot_general(g1[...], w1[:, :FEAT], dn,
                        preferred_element_type=jnp.float32)
    z += lax.dot_general(g2acc[...], w1[:, FEAT:], dn,
                         preferred_element_type=jnp.float32)
    z += b1[...]
    z = jnp.maximum(z, 0.0)
    o = lax.dot_general(z, w2[...], dn, preferred_element_type=jnp.float32)
    o += b2[...]
    mx = jnp.max(o, axis=-1, keepdims=True)
    e = jnp.exp(o - mx)
    out[...] = e / jnp.sum(e, axis=-1, keepdims=True)


def _tail(batch2d, s_lo, s_hi, t_lo, t_hi, cnt, wl, wr, bl,
          g1, w1, b1, w2, b2):
  half_spec = pl.BlockSpec((_R_TILE, HALF), lambda i: (i, 0))
  full = lambda shape: pl.BlockSpec(shape, lambda i: tuple(0 for _ in shape))
  return pl.pallas_call(
      _tail_body,
      grid=(_R_GRID,),
      in_specs=[
          pl.BlockSpec((_R_TILE, 1), lambda i: (i, 0)),
          half_spec, half_spec, half_spec, half_spec,
          pl.BlockSpec((_R_TILE, NC * NS), lambda i: (i, 0)),
          full((FEAT, FEAT)), full((FEAT, FEAT)), full((1, FEAT)),
          full((NSEG, FEAT)), full((FEAT, 2 * FEAT)), full((1, FEAT)),
          full((10, FEAT)), full((1, 10)),
      ],
      out_specs=pl.BlockSpec((NSEG, 10), lambda i: (0, 0)),
      out_shape=jax.ShapeDtypeStruct((NSEG, 10), jnp.float32),
      scratch_shapes=[pltpu.VMEM((NSEG, FEAT), jnp.float32)],
  )(batch2d, s_lo, s_hi, t_lo, t_hi, cnt, wl, wr, bl.reshape(1, FEAT),
    g1, w1, b1.reshape(1, FEAT), w2, b2.reshape(1, 10))


_AGG1 = _make_agg(fused_gather=True)
_AGG = _make_agg()
_COUNT = _make_count()


def kernel(x, edge_index, batch, emb, Wl1, bl1, Wr1, Wl2, bl2, Wr2,
           W1, b1, W2, b2):
  x = x.astype(jnp.int32)
  edge_index = edge_index.astype(jnp.int32)
  batch = batch.astype(jnp.int32)
  emb_lo = emb[:, :HALF]
  emb_hi = emb[:, HALF:]

  cnt = _COUNT(edge_index[1])
  cnt = cnt.reshape(NC * NS, N).T
  h_lo, h_hi, s1_lo, s1_hi = _AGG1(emb_lo, emb_hi, x, edge_index)
  h1_lo, h1_hi = _sage_linear(s1_lo, s1_hi, h_lo, h_hi, cnt, Wl1, Wr1, bl1)
  batch2d = batch.reshape(N, 1)
  g1 = _segmax_tc(batch2d, h1_lo, h1_hi)
  s2_lo, s2_hi = _AGG(h1_lo, h1_hi, edge_index)
  return _tail(batch2d, s2_lo, s2_hi, h1_lo, h1_hi, cnt, Wl2, Wr2, bl2,
               g1, W1, b1, W2, b2)


# speedup vs baseline: 1.0038x; 1.0038x over previous
import jax
import jax.numpy as jnp
from jax import lax
from jax.experimental import pallas as pl
from jax.experimental.pallas import tpu as pltpu
from jax.experimental.pallas import tpu_sc as plsc

N = 10000
E = 160000
FEAT = 256
HALF = 128
NSEG = 64
NC = 2
NS = 16
LANES = 16

_MESH = dict(core_axis_name="c", subcore_axis_name="s", num_cores=NC,
             num_subcores=NS)

_Z_CHUNK = 200
_Z_NCHUNK = N // _Z_CHUNK


def _foreach_row_chunk(sid, fn):
  def _j(j, _):
    c = sid + NS * j
    @pl.when(c < _Z_NCHUNK)
    def _():
      fn(c * _Z_CHUNK)
    return _
  lax.fori_loop(0, (_Z_NCHUNK + NS - 1) // NS, _j, None)


_G_CHUNK = 80
_G_NCHUNK = N // _G_CHUNK


_A_CHUNK = 128
_A_NCHUNK = E // _A_CHUNK
_A_NFULL = _A_NCHUNK // NS
_A_NTAIL = _A_NCHUNK - _A_NFULL * NS


def _make_agg(fused_gather=False):
  mesh = plsc.VectorSubcoreMesh(**_MESH)

  def body(*refs):
    if fused_gather:
      (emb_lo, emb_hi, x, edge_index, h_lo, h_hi, s_lo, s_hi,
       gidx, eb0, eb1, eb2, rows0, rows1, rows2, acc,
       isem0, isem1, isem2, gsem0, gsem1, gsem2,
       ssem0, ssem1, ssem2) = refs
    else:
      (t_lo, t_hi, edge_index, s_lo, s_hi,
       eb0, eb1, eb2, rows0, rows1, rows2, acc,
       isem0, isem1, isem2, gsem0, gsem1, gsem2,
       ssem0, ssem1, ssem2) = refs
    ebufs = (eb0, eb1, eb2)
    rows = (rows0, rows1, rows2)
    isems = (isem0, isem1, isem2)
    gsems = (gsem0, gsem1, gsem2)
    ssems = (ssem0, ssem1, ssem2)

    cid = lax.axis_index("c")
    sid = lax.axis_index("s")

    if fused_gather:
      def _do_gather(table, out):
        def _g(j, _):
          chunk = sid * 8 + j
          @pl.when(chunk < _G_NCHUNK)
          def _():
            start = chunk * _G_CHUNK
            pltpu.sync_copy(x.at[pl.ds(start, _G_CHUNK)], gidx)
            pltpu.async_copy(table.at[gidx],
                             rows0.at[pl.ds(0, _G_CHUNK)], gsem0).wait()
            pltpu.sync_copy(rows0.at[pl.ds(0, _G_CHUNK)],
                            out.at[pl.ds(start, _G_CHUNK)])
          return _
        lax.fori_loop(0, 8, _g, None)

      @pl.when(cid == 0)
      def _():
        _do_gather(emb_lo, h_lo)

      @pl.when(cid == 1)
      def _():
        _do_gather(emb_hi, h_hi)
      t_lo, t_hi = h_lo, h_hi

    def _z(i, _):
      def _zz(k, __):
        rows0[i, pl.ds(k * LANES, LANES)] = jnp.zeros((LANES,), jnp.float32)
        return __
      lax.fori_loop(0, HALF // LANES, _zz, None)
      return _
    lax.fori_loop(0, 80, _z, None)

    def _j(j, _):
      c = sid + NS * j
      @pl.when(c < N // 80)
      def _():
        pltpu.sync_copy(rows0.at[pl.ds(0, 80)], acc.at[pl.ds(c * 80, 80)])
      return _
    lax.fori_loop(0, (N // 80 + NS - 1) // NS, _j, None)

    plsc.subcore_barrier()

    def _run(table):
      def _issue_idx(j, k):
        st = (sid + NS * j) * _A_CHUNK
        pltpu.async_copy(edge_index.at[:, pl.ds(st, _A_CHUNK)],
                         ebufs[k], isems[k])

      def _issue_gather(j, k):
        pltpu.async_copy(table.at[ebufs[k].at[0]], rows[k], gsems[k])

      def _wait_idx(k):
        pltpu.make_async_copy(
            edge_index.at[:, pl.ds(0, _A_CHUNK)], ebufs[k], isems[k]).wait()

      def _wait_gather(table, k):
        pltpu.make_async_copy(table.at[ebufs[k].at[0]], rows[k],
                              gsems[k]).wait()

      def _issue_scatter(k):
        pltpu.async_copy(rows[k], acc.at[ebufs[k].at[1]], ssems[k], add=True)

      def _wait_scatter(k):
        pltpu.make_async_copy(rows[k], acc.at[ebufs[k].at[1]],
                              ssems[k]).wait()

      _issue_idx(0, 0)
      _wait_idx(0)
      _issue_gather(0, 0)
      _issue_idx(1, 1)

      def _step(jj, _):
        for k in range(3):
          j = 3 * jj + k
          _wait_gather(table, k)
          _issue_scatter(k)
          if k == 0:
            @pl.when(jj > 0)
            def _():
              _wait_scatter(2)
          else:
            _wait_scatter(k - 1)
          @pl.when(j + 2 < _A_NFULL)
          def _():
            _issue_idx(j + 2, (k + 2) % 3)
          @pl.when(j + 1 < _A_NFULL)
          def _():
            _wait_idx((k + 1) % 3)
            _issue_gather(j + 1, (k + 1) % 3)
        return _
      lax.fori_loop(0, _A_NFULL // 3, _step, None)
      _wait_scatter((_A_NFULL - 1) % 3)

      @pl.when(sid < _A_NTAIL)
      def _():
        st = (_A_NFULL * NS + sid) * _A_CHUNK
        pltpu.async_copy(edge_index.at[:, pl.ds(st, _A_CHUNK)],
                         ebufs[0], isems[0])
        _wait_idx(0)
        _issue_gather(0, 0)
        _wait_gather(table, 0)
        pltpu.sync_copy(rows[0], acc.at[ebufs[0].at[1]], add=True)

    @pl.when(cid == 0)
    def _():
      _run(t_lo)

    @pl.when(cid == 1)
    def _():
      _run(t_hi)

    plsc.subcore_barrier()

    @pl.when(cid == 0)
    def _():
      _foreach_row_chunk(sid, lambda st: pltpu.sync_copy(
          acc.at[pl.ds(st, _Z_CHUNK)], s_lo.at[pl.ds(st, _Z_CHUNK)]))
    @pl.when(cid == 1)
    def _():
      _foreach_row_chunk(sid, lambda st: pltpu.sync_copy(
          acc.at[pl.ds(st, _Z_CHUNK)], s_hi.at[pl.ds(st, _Z_CHUNK)]))

  n_out = 4 if fused_gather else 2
  scratch = ([pltpu.VMEM((_G_CHUNK,), jnp.int32)] if fused_gather else []) + [
      pltpu.VMEM((2, _A_CHUNK), jnp.int32),
      pltpu.VMEM((2, _A_CHUNK), jnp.int32),
      pltpu.VMEM((2, _A_CHUNK), jnp.int32),
      pltpu.VMEM((_A_CHUNK, HALF), jnp.float32),
      pltpu.VMEM((_A_CHUNK, HALF), jnp.float32),
      pltpu.VMEM((_A_CHUNK, HALF), jnp.float32),
      pltpu.VMEM_SHARED((N, HALF), jnp.float32),
  ] + [pltpu.SemaphoreType.DMA] * 9
  return pl.kernel(
      body,
      out_type=[jax.ShapeDtypeStruct((N, HALF), jnp.float32)] * n_out,
      mesh=mesh,
      scratch_types=scratch,
  )


_C_PER_W = E // (NC * NS)


def _make_count():
  mesh = plsc.VectorSubcoreMesh(**_MESH)

  def body(dst, cnt_out, dbuf, hist, sem):
    cid = lax.axis_index("c")
    sid = lax.axis_index("s")
    wid = cid * NS + sid

    def _zh(i, _):
      hist[pl.ds(i * LANES, LANES)] = jnp.zeros((LANES,), jnp.float32)
      return _
    lax.fori_loop(0, N // LANES, _zh, None)

    dbuf[pl.ds(_C_PER_W - 8, LANES)] = jnp.zeros((LANES,), jnp.int32)
    pltpu.sync_copy(dst.at[pl.ds(wid * _C_PER_W, _C_PER_W)],
                    dbuf.at[pl.ds(0, _C_PER_W)])

    ones16 = jnp.ones((LANES,), jnp.float32)

    def _h(i, _):
      dvec = dbuf[pl.ds(i * LANES, LANES)]
      plsc.addupdate_scatter(hist, [dvec], ones16)
      return _
    lax.fori_loop(0, _C_PER_W // LANES, _h, None)
    tvec = dbuf[pl.ds(_C_PER_W - 8, LANES)]
    mask = lax.iota(jnp.int32, LANES) < 8
    plsc.addupdate_scatter(hist, [tvec], ones16, mask=mask)

    pltpu.sync_copy(hist, cnt_out.at[pl.ds(wid * N, N)])

  return pl.kernel(
      body,
      out_type=jax.ShapeDtypeStruct((NC * NS * N,), jnp.float32),
      mesh=mesh,
      scratch_types=[
          pltpu.VMEM((_C_PER_W + 8,), jnp.int32),
          pltpu.VMEM((N,), jnp.float32),
          pltpu.SemaphoreType.DMA,
      ],
      compiler_params=pltpu.CompilerParams(needs_layout_passes=False),
  )


_S_TILE = 400
_S_GRID = N // _S_TILE


def _segmax_body(b, lo, hi, out):
  i = pl.program_id(0)

  @pl.when(i == 0)
  def _():
    out[...] = jnp.full((NSEG, FEAT), -jnp.inf, jnp.float32)

  bb = b[...]
  hc = jnp.concatenate([lo[...], hi[...]], axis=1)
  smin = jnp.min(bb)
  smax = jnp.max(bb)
  for s in range(NSEG):
    @pl.when(jnp.logical_and(smin <= s, s <= smax))
    def _():
      m = jnp.where(bb == s, hc, -jnp.inf)
      v = jnp.max(m, axis=0, keepdims=True)
      out[pl.ds(s, 1), :] = jnp.maximum(out[pl.ds(s, 1), :], v)


def _segmax_tc(batch2d, lo, hi):
  half_spec = pl.BlockSpec((_S_TILE, HALF), lambda i: (i, 0))
  return pl.pallas_call(
      _segmax_body,
      grid=(_S_GRID,),
      in_specs=[
          pl.BlockSpec((_S_TILE, 1), lambda i: (i, 0)),
          half_spec, half_spec,
      ],
      out_specs=pl.BlockSpec((NSEG, FEAT), lambda i: (0, 0)),
      out_shape=jax.ShapeDtypeStruct((NSEG, FEAT), jnp.float32),
  )(batch2d, lo, hi)


_R_TILE = 400
_R_GRID = N // _R_TILE


def _bdot(a, b):
  return lax.dot_general(a, b, (((1,), (1,)), ((), ())),
                         preferred_element_type=jnp.float32)


def _sage_dots(m_lo, m_hi, t_lo, t_hi, wlv, wrv):
  acc = _bdot(m_lo, wlv[:, :HALF])
  acc += _bdot(m_hi, wlv[:, HALF:])
  acc += _bdot(t_lo, wrv[:, :HALF])
  acc += _bdot(t_hi, wrv[:, HALF:])
  return acc


def _sage_linear_body(s_lo, s_hi, t_lo, t_hi, cnt, wl, wr, bl,
                      out_lo, out_hi):
  c = lax.dot_general(cnt[...], jnp.ones((NC * NS, 1), jnp.float32),
                      (((1,), (0,)), ((), ())),
                      preferred_element_type=jnp.float32)
  inv = 1.0 / jnp.maximum(c, 1.0)
  acc = _sage_dots(s_lo[...] * inv, s_hi[...] * inv, t_lo[...], t_hi[...],
                   wl[...], wr[...]) + bl[...]
  out_lo[...] = acc[:, :HALF]
  out_hi[...] = acc[:, HALF:]


def _sage_linear(s_lo, s_hi, t_lo, t_hi, cnt, wl, wr, bl):
  half_spec = pl.BlockSpec((_R_TILE, HALF), lambda i: (i, 0))
  return pl.pallas_call(
      _sage_linear_body,
      grid=(_R_GRID,),
      in_specs=[
          half_spec, half_spec, half_spec, half_spec,
          pl.BlockSpec((_R_TILE, NC * NS), lambda i: (i, 0)),
          pl.BlockSpec((FEAT, FEAT), lambda i: (0, 0)),
          pl.BlockSpec((FEAT, FEAT), lambda i: (0, 0)),
          pl.BlockSpec((1, FEAT), lambda i: (0, 0)),
      ],
      out_specs=[half_spec, half_spec],
      out_shape=[
          jax.ShapeDtypeStruct((N, HALF), jnp.float32),
          jax.ShapeDtypeStruct((N, HALF), jnp.float32),
      ],
  )(s_lo, s_hi, t_lo, t_hi, cnt, wl, wr, bl.reshape(1, FEAT))


def _tail_body(b, s_lo, s_hi, t_lo, t_hi, cnt, wl, wr, bl,
               g1, w1, b1, w2, b2, out, g2acc):
  i = pl.program_id(0)

  @pl.when(i == 0)
  def _():
    g2acc[...] = jnp.full((NSEG, FEAT), -jnp.inf, jnp.float32)

  c = lax.dot_general(cnt[...], jnp.ones((NC * NS, 1), jnp.float32),
                      (((1,), (0,)), ((), ())),
                      preferred_element_type=jnp.float32)
  inv = 1.0 / jnp.maximum(c, 1.0)
  dn = (((1,), (1,)), ((), ()))
  acc = _sage_dots(s_lo[...] * inv, s_hi[...] * inv, t_lo[...], t_hi[...],
                   wl[...], wr[...])
  acc += bl[...]

  bb = b[...]
  smin = jnp.min(bb)
  smax = jnp.max(bb)
  for s in range(NSEG):
    @pl.when(jnp.logical_and(smin <= s, s <= smax))
    def _():
      m = jnp.where(bb == s, acc, -jnp.inf)
      v = jnp.max(m, axis=0, keepdims=True)
      g2acc[pl.ds(s, 1), :] = jnp.maximum(g2acc[pl.ds(s, 1), :], v)

  @pl.when(i == _R_GRID - 1)
  def _():
    z = lax.dot_general(g1[...], w1[:, :FEAT], dn,
                        preferred_element_type=jnp.float32)
    z += lax.dot_general(g2acc[...], w1[:, FEAT:], dn,
                         preferred_element_type=jnp.float32)
    z += b1[...]
    z = jnp.maximum(z, 0.0)
    o = lax.dot_general(z, w2[...], dn, preferred_element_type=jnp.float32)
    o += b2[...]
    mx = jnp.max(o, axis=-1, keepdims=True)
    e = jnp.exp(o - mx)
    out[...] = e / jnp.sum(e, axis=-1, keepdims=True)


def _tail(batch2d, s_lo, s_hi, t_lo, t_hi, cnt, wl, wr, bl,
          g1, w1, b1, w2, b2):
  half_spec = pl.BlockSpec((_R_TILE, HALF), lambda i: (i, 0))
  full = lambda shape: pl.BlockSpec(shape, lambda i: tuple(0 for _ in shape))
  return pl.pallas_call(
      _tail_body,
      grid=(_R_GRID,),
      in_specs=[
          pl.BlockSpec((_R_TILE, 1), lambda i: (i, 0)),
          half_spec, half_spec, half_spec, half_spec,
          pl.BlockSpec((_R_TILE, NC * NS), lambda i: (i, 0)),
          full((FEAT, FEAT)), full((FEAT, FEAT)), full((1, FEAT)),
          full((NSEG, FEAT)), full((FEAT, 2 * FEAT)), full((1, FEAT)),
          full((10, FEAT)), full((1, 10)),
      ],
      out_specs=pl.BlockSpec((NSEG, 10), lambda i: (0, 0)),
      out_shape=jax.ShapeDtypeStruct((NSEG, 10), jnp.float32),
      scratch_shapes=[pltpu.VMEM((NSEG, FEAT), jnp.float32)],
  )(batch2d, s_lo, s_hi, t_lo, t_hi, cnt, wl, wr, bl.reshape(1, FEAT),
    g1, w1, b1.reshape(1, FEAT), w2, b2.reshape(1, 10))


_AGG1 = _make_agg(fused_gather=True)
_AGG = _make_agg()
_COUNT = _make_count()


def kernel(x, edge_index, batch, emb, Wl1, bl1, Wr1, Wl2, bl2, Wr2,
           W1, b1, W2, b2):
  x = x.astype(jnp.int32)
  edge_index = edge_index.astype(jnp.int32)
  batch = batch.astype(jnp.int32)
  emb_lo = emb[:, :HALF]
  emb_hi = emb[:, HALF:]

  cnt = _COUNT(edge_index[1])
  cnt = cnt.reshape(NC * NS, N).T
  h_lo, h_hi, s1_lo, s1_hi = _AGG1(emb_lo, emb_hi, x, edge_index)
  h1_lo, h1_hi = _sage_linear(s1_lo, s1_hi, h_lo, h_hi, cnt, Wl1, Wr1, bl1)
  batch2d = batch.reshape(N, 1)
  g1 = _segmax_tc(batch2d, h1_lo, h1_hi)
  s2_lo, s2_hi = _AGG(h1_lo, h1_hi, edge_index)
  return _tail(batch2d, s2_lo, s2_hi, h1_lo, h1_hi, cnt, Wl2, Wr2, bl2,
               g1, W1, b1, W2, b2)
